# baseline (speedup 1.0000x reference)
import functools

import jax
import jax.numpy as jnp
from jax import lax
from jax.experimental import pallas as pl
from jax.experimental.pallas import tpu as pltpu
from jax.experimental.pallas import tpu_sc as plsc

NUM_NODES = 10000
NUM_RELATIONS = 4
E_PER_REL = 80000
D_FEAT = 128
D_EDGE = 16
D_HIDDEN = 128
LN_EPS = 1e-6

NC = 2
NS = 16
CHUNK = 64
N_CHUNKS = E_PER_REL // CHUNK
MAIN_G = N_CHUNKS // NS // 2
N_TAIL = N_CHUNKS - 2 * MAIN_G * NS
N_PAD = 10240
ROWS_PER_TILE = N_PAD // NS
AW = D_FEAT + D_EDGE
DEG_COL = D_FEAT


def _fill(ref, val):
    rows, width = ref.shape

    def body(i, carry):
        for j in range(width // 16):
            ref[i, pl.ds(j * 16, 16)] = jnp.full((16,), val, ref.dtype)
        return carry

    lax.fori_loop(0, rows, body, 0)


def _sc_segment_sums(nodes_aug, senders, receivers, edges):
    mesh = plsc.VectorSubcoreMesh(core_axis_name="c", subcore_axis_name="s",
                                  num_cores=NC, num_subcores=NS)

    @functools.partial(
        pl.kernel,
        out_type=(
            jax.ShapeDtypeStruct((NUM_RELATIONS, N_PAD, AW), jnp.float32),
            jax.ShapeDtypeStruct((NC, N_PAD, D_EDGE), jnp.float32),
        ),
        mesh=mesh,
        scratch_types=[
            pltpu.VMEM_SHARED((N_PAD, AW), jnp.float32),
            pltpu.VMEM_SHARED((N_PAD, D_EDGE), jnp.float32),
            pltpu.VMEM((2, CHUNK), jnp.int32),
            pltpu.VMEM((2, CHUNK), jnp.int32),
            pltpu.VMEM((CHUNK, AW), jnp.float32),
            pltpu.VMEM((CHUNK, AW), jnp.float32),
            pltpu.VMEM((CHUNK, D_EDGE), jnp.float32),
            pltpu.VMEM((CHUNK, D_EDGE), jnp.float32),
            pltpu.SemaphoreType.DMA,
            pltpu.SemaphoreType.DMA,
            pltpu.SemaphoreType.DMA,
            pltpu.SemaphoreType.DMA,
            pltpu.SemaphoreType.DMA,
        ],
        compiler_params=pltpu.CompilerParams(use_tc_tiling_on_sc=False),
    )
    def sc_kernel(nodes_hbm, sr_hbm, edges_hbm,
                  acc_out, eacc_out,
                  acc_sh, eacc_sh, idx0, idx1, rows0, rows1,
                  er0, er1, sem0, sem1, se0, se1, sem_s):
        cid = lax.axis_index("c")
        sid = lax.axis_index("s")
        stripe = pl.ds(pl.multiple_of(sid * ROWS_PER_TILE, 8), ROWS_PER_TILE)

        for phase in range(2):
            r = cid + 2 * phase
            use_edges = phase == 0

            def load(k, idx, er, sem_e):
                pltpu.sync_copy(sr_hbm.at[r].at[k], idx)
                if use_edges:
                    base = pl.ds(pl.multiple_of(k * CHUNK, CHUNK), CHUNK)
                    pltpu.async_copy(edges_hbm.at[r].at[base], er, sem_e)

            def gather(idx, rows, sem):
                pltpu.async_copy(nodes_hbm.at[idx.at[0]], rows, sem)

            def consume(k, idx, rows, er, sem, sem_e):
                pltpu.make_async_copy(nodes_hbm.at[idx.at[0]], rows,
                                      sem).wait()
                pltpu.async_copy(rows, acc_sh.at[idx.at[1]], sem_s, add=True)
                if use_edges:
                    base = pl.ds(pl.multiple_of(k * CHUNK, CHUNK), CHUNK)
                    pltpu.make_async_copy(edges_hbm.at[r].at[base], er,
                                          sem_e).wait()
                    pltpu.async_copy(er, eacc_sh.at[idx.at[1]], sem_s,
                                     add=True)
                    pltpu.make_async_copy(er, eacc_sh.at[idx.at[1]],
                                          sem_s).wait()
                pltpu.make_async_copy(rows, acc_sh.at[idx.at[1]],
                                      sem_s).wait()

            _fill(rows0, 0.0)
            if use_edges:
                _fill(er0, 0.0)

            def zero_body(k, carry):
                dst = pl.ds(
                    pl.multiple_of(sid * ROWS_PER_TILE + k * CHUNK, 8), CHUNK)
                pltpu.sync_copy(rows0, acc_sh.at[dst])
                if use_edges:
                    pltpu.sync_copy(er0, eacc_sh.at[dst])
                return carry

            lax.fori_loop(0, ROWS_PER_TILE // CHUNK, zero_body, 0)

            load(sid, idx0, er0, se0)
            gather(idx0, rows0, sem0)
            plsc.subcore_barrier()

            def body(g, carry):
                k1 = (2 * g + 1) * NS + sid
                load(k1, idx1, er1, se1)
                gather(idx1, rows1, sem1)
                k0 = 2 * g * NS + sid
                consume(k0, idx0, rows0, er0, sem0, se0)

                @pl.when(jnp.logical_or(g < MAIN_G - 1, sid < N_TAIL))
                def _():
                    k2 = (2 * g + 2) * NS + sid
                    load(k2, idx0, er0, se0)
                    gather(idx0, rows0, sem0)

                consume(k1, idx1, rows1, er1, sem1, se1)
                return carry

            lax.fori_loop(0, MAIN_G, body, 0)

            @pl.when(sid < N_TAIL)
            def _():
                kt = 2 * MAIN_G * NS + sid
                consume(kt, idx0, rows0, er0, sem0, se0)

            plsc.subcore_barrier()
            pltpu.sync_copy(acc_sh.at[stripe], acc_out.at[r].at[stripe])
            if use_edges:
                pltpu.sync_copy(eacc_sh.at[stripe], eacc_out.at[cid].at[stripe])

    sr = jnp.stack([senders.reshape(NUM_RELATIONS, N_CHUNKS, CHUNK),
                    receivers.reshape(NUM_RELATIONS, N_CHUNKS, CHUNK)],
                   axis=2)
    return sc_kernel(nodes_aug, sr, edges)


BR = 1000


def _tc_dense(nodes, acc, eacc, W_aug, W_bot, W_node, ln_scale, ln_bias):
    def body(nodes_ref, acc_ref, eacc_ref, wa_ref, wbot_ref, wn_ref,
             g_ref, b_ref, o_ref):
        x = jnp.dot(nodes_ref[...], wn_ref[...],
                    preferred_element_type=jnp.float32)
        for r in range(NUM_RELATIONS):
            m = jnp.dot(acc_ref[r], wa_ref[r],
                        preferred_element_type=jnp.float32)
            if r < 2:
                m = m + jnp.dot(eacc_ref[r], wbot_ref[r],
                                preferred_element_type=jnp.float32)
            d = acc_ref[r, :, DEG_COL:DEG_COL + 1]
            x = x + m * lax.reciprocal(jnp.maximum(d, 1.0))
        mean = jnp.mean(x, axis=-1, keepdims=True)
        var = jnp.mean(jnp.square(x - mean), axis=-1, keepdims=True)
        x = (x - mean) * lax.rsqrt(var + LN_EPS) * g_ref[...] + b_ref[...]
        o_ref[...] = jnp.maximum(x, 0.0)

    grid = (NUM_NODES // BR,)
    return pl.pallas_call(
        body,
        grid=grid,
        in_specs=[
            pl.BlockSpec((BR, D_FEAT), lambda i: (i, 0)),
            pl.BlockSpec((NUM_RELATIONS, BR, AW), lambda i: (0, i, 0)),
            pl.BlockSpec((NC, BR, D_EDGE), lambda i: (0, i, 0)),
            pl.BlockSpec((NUM_RELATIONS, AW, D_HIDDEN), lambda i: (0, 0, 0)),
            pl.BlockSpec((NC, D_EDGE, D_HIDDEN), lambda i: (0, 0, 0)),
            pl.BlockSpec((D_FEAT, D_HIDDEN), lambda i: (0, 0)),
            pl.BlockSpec((1, D_HIDDEN), lambda i: (0, 0)),
            pl.BlockSpec((1, D_HIDDEN), lambda i: (0, 0)),
        ],
        out_specs=pl.BlockSpec((BR, D_HIDDEN), lambda i: (i, 0)),
        out_shape=jax.ShapeDtypeStruct((NUM_NODES, D_HIDDEN), jnp.float32),
    )(nodes, acc, eacc, W_aug, W_bot, W_node,
      ln_scale.reshape(1, D_HIDDEN), ln_bias.reshape(1, D_HIDDEN))


def kernel(nodes, edges, senders, receivers, W_node, W_rel0, W_rel1, W_rel2,
           W_rel3, ln_scale, ln_bias):
    nodes_aug = jnp.concatenate(
        [nodes, jnp.ones((NUM_NODES, D_EDGE), jnp.float32)], axis=1)
    acc, eacc = _sc_segment_sums(nodes_aug, senders, receivers, edges)
    zpad = jnp.zeros((D_EDGE, D_HIDDEN), jnp.float32)
    W_aug = jnp.stack([
        jnp.concatenate([W_rel0[:D_FEAT], zpad], axis=0),
        jnp.concatenate([W_rel1[:D_FEAT], zpad], axis=0),
        jnp.concatenate([W_rel2, zpad], axis=0),
        jnp.concatenate([W_rel3, zpad], axis=0)], axis=0)
    W_bot = jnp.stack([W_rel0[D_FEAT:], W_rel1[D_FEAT:]], axis=0)
    return _tc_dense(nodes, acc, eacc, W_aug, W_bot, W_node,
                     ln_scale, ln_bias)


# speedup vs baseline: 1.1662x; 1.1662x over previous
import functools

import jax
import jax.numpy as jnp
from jax import lax
from jax.experimental import pallas as pl
from jax.experimental.pallas import tpu as pltpu
from jax.experimental.pallas import tpu_sc as plsc

NUM_NODES = 10000
NUM_RELATIONS = 4
E_PER_REL = 80000
D_FEAT = 128
D_EDGE = 16
D_HIDDEN = 128
LN_EPS = 1e-6

NC = 2
NS = 16
CHUNK = 64
N_CHUNKS = E_PER_REL // CHUNK
N_FULL = N_CHUNKS // NS
MAIN_G = N_FULL // 3
N_TAIL = N_CHUNKS - N_FULL * NS
N_PAD = 10112
ROWS_PER_TILE = N_PAD // NS
ZFULL = ROWS_PER_TILE // CHUNK
ZREM = ROWS_PER_TILE - ZFULL * CHUNK
DEG_W = 16


def _fill(ref, val):
    rows, width = ref.shape

    def body(i, carry):
        for j in range(width // 16):
            ref[i, pl.ds(j * 16, 16)] = jnp.full((16,), val, ref.dtype)
        return carry

    lax.fori_loop(0, rows, body, 0)


def _sc_segment_sums(nodes, senders, receivers, edges):
    mesh = plsc.VectorSubcoreMesh(core_axis_name="c", subcore_axis_name="s",
                                  num_cores=NC, num_subcores=NS)

    @functools.partial(
        pl.kernel,
        out_type=(
            jax.ShapeDtypeStruct((NUM_RELATIONS, N_PAD, D_FEAT), jnp.float32),
            jax.ShapeDtypeStruct((NC, N_PAD, D_EDGE), jnp.float32),
            jax.ShapeDtypeStruct((NUM_RELATIONS, N_PAD, DEG_W), jnp.float32),
        ),
        mesh=mesh,
        scratch_types=[
            pltpu.VMEM_SHARED((N_PAD, D_FEAT), jnp.float32),
            pltpu.VMEM_SHARED((N_PAD, D_EDGE), jnp.float32),
            pltpu.VMEM_SHARED((N_PAD, DEG_W), jnp.float32),
            pltpu.VMEM((2, CHUNK), jnp.int32),
            pltpu.VMEM((2, CHUNK), jnp.int32),
            pltpu.VMEM((2, CHUNK), jnp.int32),
            pltpu.VMEM((CHUNK, D_FEAT), jnp.float32),
            pltpu.VMEM((CHUNK, D_FEAT), jnp.float32),
            pltpu.VMEM((CHUNK, D_FEAT), jnp.float32),
            pltpu.VMEM((CHUNK, D_EDGE), jnp.float32),
            pltpu.VMEM((CHUNK, D_EDGE), jnp.float32),
            pltpu.VMEM((CHUNK, D_EDGE), jnp.float32),
            pltpu.VMEM((CHUNK, DEG_W), jnp.float32),
            pltpu.SemaphoreType.DMA,
            pltpu.SemaphoreType.DMA,
            pltpu.SemaphoreType.DMA,
            pltpu.SemaphoreType.DMA,
            pltpu.SemaphoreType.DMA,
            pltpu.SemaphoreType.DMA,
            pltpu.SemaphoreType.DMA,
            pltpu.SemaphoreType.DMA,
            pltpu.SemaphoreType.DMA,
            pltpu.SemaphoreType.DMA,
        ],
        compiler_params=pltpu.CompilerParams(use_tc_tiling_on_sc=False),
    )
    def sc_kernel(nodes_hbm, sr_hbm, edges_hbm,
                  acc_out, eacc_out, deg_out,
                  acc_sh, eacc_sh, deg_sh, idx0, idx1, idx2,
                  rows0, rows1, rows2, er0, er1, er2, ones_v,
                  semi0, semi1, semi2, semg0, semg1, semg2,
                  seme0, seme1, seme2, sem_s):
        cid = lax.axis_index("c")
        sid = lax.axis_index("s")
        stripe = pl.ds(pl.multiple_of(sid * ROWS_PER_TILE, 8), ROWS_PER_TILE)
        idxs = (idx0, idx1, idx2)
        rows = (rows0, rows1, rows2)
        ers = (er0, er1, er2)
        semi = (semi0, semi1, semi2)
        semg = (semg0, semg1, semg2)
        seme = (seme0, seme1, seme2)
        _fill(ones_v, 1.0)

        for phase in range(2):
            r = cid + 2 * phase
            use_edges = phase == 0

            def valid(i):
                return jnp.logical_or(
                    i < N_FULL,
                    jnp.logical_and(i == N_FULL, sid < N_TAIL))

            def start_iload(i, b):
                k = i * NS + sid
                pltpu.async_copy(sr_hbm.at[r].at[k], idxs[b], semi[b])

            def start_eload(i, b):
                k = i * NS + sid
                base = pl.ds(pl.multiple_of(k * CHUNK, CHUNK), CHUNK)
                pltpu.async_copy(edges_hbm.at[r].at[base], ers[b], seme[b])

            def issue_gather(i, b):
                k = i * NS + sid
                pltpu.make_async_copy(sr_hbm.at[r].at[k], idxs[b],
                                      semi[b]).wait()
                pltpu.async_copy(nodes_hbm.at[idxs[b].at[0]], rows[b],
                                 semg[b])

            def consume(i, b):
                k = i * NS + sid
                pltpu.make_async_copy(nodes_hbm.at[idxs[b].at[0]], rows[b],
                                      semg[b]).wait()
                pltpu.async_copy(rows[b], acc_sh.at[idxs[b].at[1]], sem_s,
                                 add=True)
                pltpu.async_copy(ones_v, deg_sh.at[idxs[b].at[1]], sem_s,
                                 add=True)
                if use_edges:
                    base = pl.ds(pl.multiple_of(k * CHUNK, CHUNK), CHUNK)
                    pltpu.make_async_copy(edges_hbm.at[r].at[base], ers[b],
                                          seme[b]).wait()
                    pltpu.async_copy(ers[b], eacc_sh.at[idxs[b].at[1]],
                                     sem_s, add=True)
                    pltpu.make_async_copy(ers[b], eacc_sh.at[idxs[b].at[1]],
                                          sem_s).wait()
                pltpu.make_async_copy(rows[b], acc_sh.at[idxs[b].at[1]],
                                      sem_s).wait()
                pltpu.make_async_copy(ones_v, deg_sh.at[idxs[b].at[1]],
                                      sem_s).wait()

            _fill(rows0, 0.0)
            _fill(er0, 0.0)

            def zero_body(z, carry):
                dst = pl.ds(
                    pl.multiple_of(sid * ROWS_PER_TILE + z * CHUNK, 8), CHUNK)
                pltpu.sync_copy(rows0, acc_sh.at[dst])
                pltpu.sync_copy(er0, deg_sh.at[dst])
                if use_edges:
                    pltpu.sync_copy(er0, eacc_sh.at[dst])
                return carry

            lax.fori_loop(0, ZFULL, zero_body, 0)
            zdst = pl.ds(
                pl.multiple_of(sid * ROWS_PER_TILE + ZFULL * CHUNK, 8), ZREM)
            pltpu.sync_copy(rows0.at[pl.ds(0, ZREM)], acc_sh.at[zdst])
            pltpu.sync_copy(er0.at[pl.ds(0, ZREM)], deg_sh.at[zdst])
            if use_edges:
                pltpu.sync_copy(er0.at[pl.ds(0, ZREM)], eacc_sh.at[zdst])

            start_iload(0, 0)
            start_iload(1, 1)
            start_iload(2, 2)
            if use_edges:
                start_eload(0, 0)
            issue_gather(0, 0)
            issue_gather(1, 1)
            plsc.subcore_barrier()

            def body(g, carry):
                for j in range(3):
                    i = 3 * g + j

                    @pl.when(valid(i + 2))
                    def _():
                        issue_gather(i + 2, (j + 2) % 3)

                    if use_edges:
                        @pl.when(valid(i + 1))
                        def _():
                            start_eload(i + 1, (j + 1) % 3)

                    consume(i, j)

                    @pl.when(valid(i + 3))
                    def _():
                        start_iload(i + 3, j)
                return carry

            lax.fori_loop(0, MAIN_G, body, 0)

            @pl.when(sid < N_TAIL)
            def _():
                consume(N_FULL, N_FULL % 3)

            plsc.subcore_barrier()
            pltpu.sync_copy(acc_sh.at[stripe], acc_out.at[r].at[stripe])
            pltpu.sync_copy(deg_sh.at[stripe], deg_out.at[r].at[stripe])
            if use_edges:
                pltpu.sync_copy(eacc_sh.at[stripe], eacc_out.at[cid].at[stripe])

    sr = jnp.stack([senders.reshape(NUM_RELATIONS, N_CHUNKS, CHUNK),
                    receivers.reshape(NUM_RELATIONS, N_CHUNKS, CHUNK)],
                   axis=2)
    return sc_kernel(nodes, sr, edges)


BR = 1000


def _tc_dense(nodes, acc, eacc, deg, W_node, W_rel0, W_rel1, W_rel2, W_rel3,
              ln_scale, ln_bias):
    def body(nodes_ref, acc_ref, eacc_ref, deg_ref, wn_ref, w0_ref, w1_ref,
             w2_ref, w3_ref, g_ref, b_ref, o_ref):
        x = jnp.dot(nodes_ref[...], wn_ref[...],
                    preferred_element_type=jnp.float32)
        w_refs = (w0_ref, w1_ref, w2_ref, w3_ref)
        for r in range(NUM_RELATIONS):
            m = jnp.dot(acc_ref[r], w_refs[r][0:D_FEAT, :],
                        preferred_element_type=jnp.float32)
            if r < 2:
                m = m + jnp.dot(eacc_ref[r], w_refs[r][D_FEAT:D_FEAT + D_EDGE, :],
                                preferred_element_type=jnp.float32)
            d = deg_ref[r, :, 0:1]
            x = x + m * lax.reciprocal(jnp.maximum(d, 1.0))
        mean = jnp.mean(x, axis=-1, keepdims=True)
        var = jnp.mean(jnp.square(x - mean), axis=-1, keepdims=True)
        x = (x - mean) * lax.rsqrt(var + LN_EPS) * g_ref[...] + b_ref[...]
        o_ref[...] = jnp.maximum(x, 0.0)

    grid = (NUM_NODES // BR,)
    return pl.pallas_call(
        body,
        grid=grid,
        in_specs=[
            pl.BlockSpec((BR, D_FEAT), lambda i: (i, 0)),
            pl.BlockSpec((NUM_RELATIONS, BR, D_FEAT), lambda i: (0, i, 0)),
            pl.BlockSpec((NC, BR, D_EDGE), lambda i: (0, i, 0)),
            pl.BlockSpec((NUM_RELATIONS, BR, DEG_W), lambda i: (0, i, 0)),
            pl.BlockSpec((D_FEAT, D_HIDDEN), lambda i: (0, 0)),
            pl.BlockSpec((D_FEAT + D_EDGE, D_HIDDEN), lambda i: (0, 0)),
            pl.BlockSpec((D_FEAT + D_EDGE, D_HIDDEN), lambda i: (0, 0)),
            pl.BlockSpec((D_FEAT, D_HIDDEN), lambda i: (0, 0)),
            pl.BlockSpec((D_FEAT, D_HIDDEN), lambda i: (0, 0)),
            pl.BlockSpec((1, D_HIDDEN), lambda i: (0, 0)),
            pl.BlockSpec((1, D_HIDDEN), lambda i: (0, 0)),
        ],
        out_specs=pl.BlockSpec((BR, D_HIDDEN), lambda i: (i, 0)),
        out_shape=jax.ShapeDtypeStruct((NUM_NODES, D_HIDDEN), jnp.float32),
    )(nodes, acc, eacc, deg, W_node, W_rel0, W_rel1, W_rel2, W_rel3,
      ln_scale.reshape(1, D_HIDDEN), ln_bias.reshape(1, D_HIDDEN))


def kernel(nodes, edges, senders, receivers, W_node, W_rel0, W_rel1, W_rel2,
           W_rel3, ln_scale, ln_bias):
    acc, eacc, deg = _sc_segment_sums(nodes, senders, receivers, edges)
    return _tc_dense(nodes, acc, eacc, deg, W_node, W_rel0, W_rel1, W_rel2,
                     W_rel3, ln_scale, ln_bias)
